# Initial kernel scaffold; baseline (speedup 1.0000x reference)
import functools

import jax
import jax.numpy as jnp
from jax import lax
from jax.experimental import pallas as pl
from jax.experimental.pallas import tpu as pltpu
from jax.experimental.pallas import tpu_sc as plsc

_N = 10000
_E = 320000
_H = 128
_D = 32
_G = 64

_NC = 2
_NS = 16
_NW = _NC * _NS
_EPT = _E // _NW
_CE = 80
_NCH = _EPT // _CE
_NPT = _N // _NS

_BN = 1000
_BE = 2000

_LP = (1.472065010832413e-06, 0.999847697496239, -0.4973732161580147,
       0.3157473167582865, -0.19035433673370444, 0.08269123711198781,
       -0.017414077524446427)


def _log1p_poly(t):
  acc = jnp.full_like(t, _LP[6])
  for c in (_LP[5], _LP[4], _LP[3], _LP[2], _LP[1], _LP[0]):
    acc = acc * t + c
  return acc


def _cprep_body(ea_ref, we_ref, be_ref, c1_ref, c2_ref, c3_ref):
  c = jnp.dot(ea_ref[...], we_ref[...],
              preferred_element_type=jnp.float32) + be_ref[...]
  c1_ref[...] = c[:, 0:256]
  c2_ref[...] = c[:, 256:512]
  c3_ref[...] = c[:, 512:768]


_cprep = pl.pallas_call(
    _cprep_body,
    grid=(_E // _BE,),
    in_specs=[
        pl.BlockSpec((_BE, _D), lambda i: (i, 0)),
        pl.BlockSpec((_D, 768), lambda i: (0, 0)),
        pl.BlockSpec((1, 768), lambda i: (0, 0)),
    ],
    out_specs=[
        pl.BlockSpec((_BE, 2 * _H), lambda i: (i, 0)),
        pl.BlockSpec((_BE, 2 * _H), lambda i: (i, 0)),
        pl.BlockSpec((_BE, 2 * _H), lambda i: (i, 0)),
    ],
    out_shape=[jax.ShapeDtypeStruct((_E, 2 * _H), jnp.float32)] * 3,
)


def _proj1_body(x_ref, emb_ref, wd_ref, ws_ref, h_ref, ad_ref, as_ref):
  iot = lax.broadcasted_iota(jnp.int32, (_BN, _H), 1)
  oh = (x_ref[...] == iot).astype(jnp.float32)
  h = jnp.dot(oh, emb_ref[...], preferred_element_type=jnp.float32)
  h_ref[...] = h
  ad_ref[...] = jnp.dot(h, wd_ref[...], preferred_element_type=jnp.float32)
  as_ref[...] = jnp.dot(h, ws_ref[...], preferred_element_type=jnp.float32)


_proj1 = pl.pallas_call(
    _proj1_body,
    grid=(_N // _BN,),
    in_specs=[
        pl.BlockSpec((_BN, 1), lambda i: (i, 0)),
        pl.BlockSpec((_H, _H), lambda i: (0, 0)),
        pl.BlockSpec((_H, 2 * _H), lambda i: (0, 0)),
        pl.BlockSpec((_H, 2 * _H), lambda i: (0, 0)),
    ],
    out_specs=[
        pl.BlockSpec((_BN, _H), lambda i: (i, 0)),
        pl.BlockSpec((_BN, 2 * _H), lambda i: (i, 0)),
        pl.BlockSpec((_BN, 2 * _H), lambda i: (i, 0)),
    ],
    out_shape=[
        jax.ShapeDtypeStruct((_N, _H), jnp.float32),
        jax.ShapeDtypeStruct((_N, 2 * _H), jnp.float32),
        jax.ShapeDtypeStruct((_N, 2 * _H), jnp.float32),
    ],
)


def _proj_body(h_ref, p_ref, wd_ref, ws_ref, hn_ref, ad_ref, as_ref):
  hn = jnp.maximum(h_ref[...] + p_ref[0] + p_ref[1], 0.0)
  hn_ref[...] = hn
  ad_ref[...] = jnp.dot(hn, wd_ref[...], preferred_element_type=jnp.float32)
  as_ref[...] = jnp.dot(hn, ws_ref[...], preferred_element_type=jnp.float32)


_proj = pl.pallas_call(
    _proj_body,
    grid=(_N // _BN,),
    in_specs=[
        pl.BlockSpec((_BN, _H), lambda i: (i, 0)),
        pl.BlockSpec((_NC, _BN, _H), lambda i: (0, i, 0)),
        pl.BlockSpec((_H, 2 * _H), lambda i: (0, 0)),
        pl.BlockSpec((_H, 2 * _H), lambda i: (0, 0)),
    ],
    out_specs=[
        pl.BlockSpec((_BN, _H), lambda i: (i, 0)),
        pl.BlockSpec((_BN, 2 * _H), lambda i: (i, 0)),
        pl.BlockSpec((_BN, 2 * _H), lambda i: (i, 0)),
    ],
    out_shape=[
        jax.ShapeDtypeStruct((_N, _H), jnp.float32),
        jax.ShapeDtypeStruct((_N, 2 * _H), jnp.float32),
        jax.ShapeDtypeStruct((_N, 2 * _H), jnp.float32),
    ],
)


def _pool_body(h_ref, p_ref, b_ref, wl_ref, bl_ref, o_ref, acc, cnt):
  i = pl.program_id(0)

  @pl.when(i == 0)
  def _():
    acc[...] = jnp.zeros_like(acc)
    cnt[...] = jnp.zeros_like(cnt)

  h3 = jnp.maximum(h_ref[...] + p_ref[0] + p_ref[1], 0.0)
  iot = lax.broadcasted_iota(jnp.int32, (_G, _BN), 0)
  pt = (b_ref[...] == iot).astype(jnp.float32)
  acc[...] += jnp.dot(pt, h3, preferred_element_type=jnp.float32)
  cnt[...] += jnp.dot(pt, jnp.ones_like(h3), preferred_element_type=jnp.float32)

  @pl.when(i == pl.num_programs(0) - 1)
  def _():
    pooled = acc[...] / jnp.maximum(cnt[...], 1.0)
    o_ref[...] = jnp.dot(pooled, wl_ref[...],
                         preferred_element_type=jnp.float32) + bl_ref[...]


_pool = pl.pallas_call(
    _pool_body,
    grid=(_N // _BN,),
    in_specs=[
        pl.BlockSpec((_BN, _H), lambda i: (i, 0)),
        pl.BlockSpec((_NC, _BN, _H), lambda i: (0, i, 0)),
        pl.BlockSpec((1, _BN), lambda i: (0, i)),
        pl.BlockSpec((_H, _H), lambda i: (0, 0)),
        pl.BlockSpec((1, _H), lambda i: (0, 0)),
    ],
    out_specs=pl.BlockSpec((_G, _H), lambda i: (0, 0)),
    out_shape=jax.ShapeDtypeStruct((_G, _H), jnp.float32),
    scratch_shapes=[
        pltpu.VMEM((_G, _H), jnp.float32),
        pltpu.VMEM((_G, _H), jnp.float32),
    ],
)


def _edge_body(ad_hbm, as_hbm, c_hbm, dst_hbm, src_hbm, zero_hbm, out_hbm,
               agg, dstb, srcb, bb, mb, sem):
  cid = lax.axis_index("c")
  sid = lax.axis_index("s")
  wid = sid * _NC + cid
  base = wid * _EPT

  pltpu.sync_copy(zero_hbm, agg.at[pl.ds(sid * _NPT, _NPT)])
  plsc.subcore_barrier()

  def chunk(i, carry):
    off = base + i * _CE
    pltpu.sync_copy(dst_hbm.at[pl.ds(off, _CE)], dstb)
    pltpu.sync_copy(src_hbm.at[pl.ds(off, _CE)], srcb)
    pltpu.sync_copy(c_hbm.at[pl.ds(off, _CE)], bb)
    pltpu.async_copy(ad_hbm.at[dstb], bb, sem, add=True).wait()
    pltpu.async_copy(as_hbm.at[srcb], bb, sem, add=True).wait()

    def edge(e, c2):
      for k in range(_H // 16):
        f = bb[e, pl.ds(k * 16, 16)]
        s = bb[e, pl.ds(_H + k * 16, 16)]
        sig = 1.0 / (1.0 + jnp.exp(-f))
        t = jnp.exp(-jnp.abs(s))
        sp = jnp.maximum(s, 0.0) + _log1p_poly(t)
        mb[e, pl.ds(k * 16, 16)] = sig * sp
      return c2

    lax.fori_loop(0, _CE, edge, 0)
    pltpu.sync_copy(mb, agg.at[dstb], add=True)
    return carry

  lax.fori_loop(0, _NCH, chunk, 0)
  plsc.subcore_barrier()
  pltpu.sync_copy(agg.at[pl.ds(sid * _NPT, _NPT)],
                  out_hbm.at[cid, pl.ds(sid * _NPT, _NPT)])


_edge_pass = functools.partial(
    pl.kernel,
    out_type=jax.ShapeDtypeStruct((_NC, _N, _H), jnp.float32),
    mesh=plsc.VectorSubcoreMesh(core_axis_name="c", subcore_axis_name="s"),
    scratch_types=[
        pltpu.VMEM_SHARED((_N, _H), jnp.float32),
        pltpu.VMEM((_CE,), jnp.int32),
        pltpu.VMEM((_CE,), jnp.int32),
        pltpu.VMEM((_CE, 2 * _H), jnp.float32),
        pltpu.VMEM((_CE, _H), jnp.float32),
        pltpu.SemaphoreType.DMA,
    ],
)(_edge_body)


def kernel(x, edge_index, edge_attr, batch, emb,
           Wf1, bf1, Ws1, bs1, Wf2, bf2, Ws2, bs2, Wf3, bf3, Ws3, bs3,
           Wlin, blin):
  f32 = jnp.float32
  x2 = x.astype(jnp.int32).reshape(_N, 1)
  src = edge_index[0].astype(jnp.int32)
  dst = edge_index[1].astype(jnp.int32)
  b2 = batch.astype(jnp.int32).reshape(1, _N)
  emb_pad = jnp.zeros((_H, _H), f32).at[:emb.shape[0]].set(emb)
  zeros = jnp.zeros((_NPT, _H), f32)

  def parts(Wf, Ws):
    wd = jnp.concatenate([Wf[:_H], Ws[:_H]], axis=1)
    wsr = jnp.concatenate([Wf[_H:2 * _H], Ws[_H:2 * _H]], axis=1)
    we = jnp.concatenate([Wf[2 * _H:], Ws[2 * _H:]], axis=1)
    return wd, wsr, we

  wd1, wsr1, we1 = parts(Wf1, Ws1)
  wd2, wsr2, we2 = parts(Wf2, Ws2)
  wd3, wsr3, we3 = parts(Wf3, Ws3)
  we_all = jnp.concatenate([we1, we2, we3], axis=1)
  be_all = jnp.concatenate([
      jnp.concatenate([bf1, bs1]),
      jnp.concatenate([bf2, bs2]),
      jnp.concatenate([bf3, bs3]),
  ]).reshape(1, 768)

  c1, c2, c3 = _cprep(edge_attr, we_all, be_all)

  h0, ad, as_ = _proj1(x2, emb_pad, wd1, wsr1)
  p = _edge_pass(ad, as_, c1, dst, src, zeros)
  h1, ad, as_ = _proj(h0, p, wd2, wsr2)
  p = _edge_pass(ad, as_, c2, dst, src, zeros)
  h2, ad, as_ = _proj(h1, p, wd3, wsr3)
  p = _edge_pass(ad, as_, c3, dst, src, zeros)
  return _pool(h2, p, b2, Wlin, blin.reshape(1, _H))


# speedup vs baseline: 1.6711x; 1.6711x over previous
import functools

import jax
import jax.numpy as jnp
from jax import lax
from jax.experimental import pallas as pl
from jax.experimental.pallas import tpu as pltpu
from jax.experimental.pallas import tpu_sc as plsc

_N = 10000
_E = 320000
_H = 128
_D = 32
_G = 64

_NC = 2
_NS = 16
_NW = _NC * _NS
_EPT = _E // _NW
_CE = 80
_NCH = _EPT // _CE
_NPAD = 10240
_NPT = _NPAD // _NS

_BN = 1000
_BE = 2000

_LP = (1.472065010832413e-06, 0.999847697496239, -0.4973732161580147,
       0.3157473167582865, -0.19035433673370444, 0.08269123711198781,
       -0.017414077524446427)


def _log1p_poly(t):
  acc = jnp.full_like(t, _LP[6])
  for c in (_LP[5], _LP[4], _LP[3], _LP[2], _LP[1], _LP[0]):
    acc = acc * t + c
  return acc


def _cprep_body(ea_ref, we_ref, be_ref, *c_refs):
  c = jnp.dot(ea_ref[...], we_ref[...],
              preferred_element_type=jnp.float32) + be_ref[...]
  for j in range(6):
    c_refs[j][...] = c[:, j * _H:(j + 1) * _H]


_cprep = pl.pallas_call(
    _cprep_body,
    grid=(_E // _BE,),
    in_specs=[
        pl.BlockSpec((_BE, _D), lambda i: (i, 0)),
        pl.BlockSpec((_D, 6 * _H), lambda i: (0, 0)),
        pl.BlockSpec((1, 6 * _H), lambda i: (0, 0)),
    ],
    out_specs=[pl.BlockSpec((_BE, _H), lambda i: (i, 0))] * 6,
    out_shape=[jax.ShapeDtypeStruct((_E, _H), jnp.float32)] * 6,
)


def _proj1_body(x_ref, emb_ref, w_ref, h_ref, *p_refs):
  iot = lax.broadcasted_iota(jnp.int32, (_BN, _H), 1)
  oh = (x_ref[...] == iot).astype(jnp.float32)
  h = jnp.dot(oh, emb_ref[...], preferred_element_type=jnp.float32)
  h_ref[...] = h
  r = jnp.dot(h, w_ref[...], preferred_element_type=jnp.float32)
  for j in range(4):
    p_refs[j][...] = r[:, j * _H:(j + 1) * _H]


_proj1 = pl.pallas_call(
    _proj1_body,
    grid=(_N // _BN,),
    in_specs=[
        pl.BlockSpec((_BN, 1), lambda i: (i, 0)),
        pl.BlockSpec((_H, _H), lambda i: (0, 0)),
        pl.BlockSpec((_H, 4 * _H), lambda i: (0, 0)),
    ],
    out_specs=[pl.BlockSpec((_BN, _H), lambda i: (i, 0))] * 5,
    out_shape=[jax.ShapeDtypeStruct((_N, _H), jnp.float32)] * 5,
)


def _proj_body(h_ref, p_ref, w_ref, hn_ref, *p_refs):
  hn = jnp.maximum(h_ref[...] + p_ref[0] + p_ref[1], 0.0)
  hn_ref[...] = hn
  r = jnp.dot(hn, w_ref[...], preferred_element_type=jnp.float32)
  for j in range(4):
    p_refs[j][...] = r[:, j * _H:(j + 1) * _H]


_proj = pl.pallas_call(
    _proj_body,
    grid=(_N // _BN,),
    in_specs=[
        pl.BlockSpec((_BN, _H), lambda i: (i, 0)),
        pl.BlockSpec((_NC, _BN, _H), lambda i: (0, i, 0)),
        pl.BlockSpec((_H, 4 * _H), lambda i: (0, 0)),
    ],
    out_specs=[pl.BlockSpec((_BN, _H), lambda i: (i, 0))] * 5,
    out_shape=[jax.ShapeDtypeStruct((_N, _H), jnp.float32)] * 5,
)


def _pool_body(h_ref, p_ref, b_ref, wl_ref, bl_ref, o_ref, acc, cnt):
  i = pl.program_id(0)

  @pl.when(i == 0)
  def _():
    acc[...] = jnp.zeros_like(acc)
    cnt[...] = jnp.zeros_like(cnt)

  h3 = jnp.maximum(h_ref[...] + p_ref[0] + p_ref[1], 0.0)
  iot = lax.broadcasted_iota(jnp.int32, (_G, _BN), 0)
  pt = (b_ref[0] == iot).astype(jnp.float32)
  acc[...] += jnp.dot(pt, h3, preferred_element_type=jnp.float32)
  cnt[...] += jnp.dot(pt, jnp.ones_like(h3), preferred_element_type=jnp.float32)

  @pl.when(i == pl.num_programs(0) - 1)
  def _():
    pooled = acc[...] / jnp.maximum(cnt[...], 1.0)
    o_ref[...] = jnp.dot(pooled, wl_ref[...],
                         preferred_element_type=jnp.float32) + bl_ref[...]


_pool = pl.pallas_call(
    _pool_body,
    grid=(_N // _BN,),
    in_specs=[
        pl.BlockSpec((_BN, _H), lambda i: (i, 0)),
        pl.BlockSpec((_NC, _BN, _H), lambda i: (0, i, 0)),
        pl.BlockSpec((1, 1, _BN), lambda i: (i, 0, 0)),
        pl.BlockSpec((_H, _H), lambda i: (0, 0)),
        pl.BlockSpec((1, _H), lambda i: (0, 0)),
    ],
    out_specs=pl.BlockSpec((_G, _H), lambda i: (0, 0)),
    out_shape=jax.ShapeDtypeStruct((_G, _H), jnp.float32),
    scratch_shapes=[
        pltpu.VMEM((_G, _H), jnp.float32),
        pltpu.VMEM((_G, _H), jnp.float32),
    ],
)


def _edge_body(pdf_hbm, pds_hbm, psf_hbm, pss_hbm, cf_hbm, cs_hbm,
               dst_hbm, src_hbm, zero_hbm, out_hbm,
               agg, dstb, srcb, bbf, bbs, mb, sem):
  cid = lax.axis_index("c")
  sid = lax.axis_index("s")
  wid = sid * _NC + cid
  base = wid * _EPT

  pltpu.sync_copy(zero_hbm, agg.at[pl.ds(sid * _NPT, _NPT)])
  plsc.subcore_barrier()

  def chunk(i, carry):
    off = base + i * _CE
    pltpu.sync_copy(dst_hbm.at[pl.ds(off, _CE)], dstb)
    pltpu.sync_copy(src_hbm.at[pl.ds(off, _CE)], srcb)
    pltpu.sync_copy(cf_hbm.at[pl.ds(off, _CE)], bbf)
    pltpu.sync_copy(cs_hbm.at[pl.ds(off, _CE)], bbs)
    pltpu.async_copy(pdf_hbm.at[dstb], bbf, sem, add=True).wait()
    pltpu.async_copy(psf_hbm.at[srcb], bbf, sem, add=True).wait()
    pltpu.async_copy(pds_hbm.at[dstb], bbs, sem, add=True).wait()
    pltpu.async_copy(pss_hbm.at[srcb], bbs, sem, add=True).wait()

    def edge(e, c2):
      for k in range(_H // 16):
        f = bbf[e, pl.ds(k * 16, 16)]
        s = bbs[e, pl.ds(k * 16, 16)]
        sig = 1.0 / (1.0 + jnp.exp(-f))
        t = jnp.exp(-jnp.abs(s))
        sp = jnp.maximum(s, 0.0) + _log1p_poly(t)
        mb[e, pl.ds(k * 16, 16)] = sig * sp
      return c2

    lax.fori_loop(0, _CE, edge, 0)
    pltpu.sync_copy(mb, agg.at[dstb], add=True)
    return carry

  lax.fori_loop(0, _NCH, chunk, 0)
  plsc.subcore_barrier()
  pltpu.sync_copy(agg.at[pl.ds(sid * _NPT, _NPT)],
                  out_hbm.at[cid, pl.ds(sid * _NPT, _NPT)])


@functools.cache
def _edge_pass_fn():
  return functools.partial(
      pl.kernel,
      out_type=jax.ShapeDtypeStruct((_NC, _NPAD, _H), jnp.float32),
      mesh=plsc.VectorSubcoreMesh(core_axis_name="c", subcore_axis_name="s",
                                  num_cores=_NC, num_subcores=_NS),
      scratch_types=[
          pltpu.VMEM_SHARED((_NPAD, _H), jnp.float32),
          pltpu.VMEM((_CE,), jnp.int32),
          pltpu.VMEM((_CE,), jnp.int32),
          pltpu.VMEM((_CE, _H), jnp.float32),
          pltpu.VMEM((_CE, _H), jnp.float32),
          pltpu.VMEM((_CE, _H), jnp.float32),
          pltpu.SemaphoreType.DMA,
      ],
  )(_edge_body)


def _edge_pass(*args):
  return _edge_pass_fn()(*args)


def kernel(x, edge_index, edge_attr, batch, emb,
           Wf1, bf1, Ws1, bs1, Wf2, bf2, Ws2, bs2, Wf3, bf3, Ws3, bs3,
           Wlin, blin):
  f32 = jnp.float32
  x2 = x.astype(jnp.int32).reshape(_N, 1)
  src = edge_index[0].astype(jnp.int32)
  dst = edge_index[1].astype(jnp.int32)
  b2 = batch.astype(jnp.int32).reshape(_N // _BN, 1, _BN)
  emb_pad = jnp.zeros((_H, _H), f32).at[:emb.shape[0]].set(emb)
  zeros = jnp.zeros((_NPT, _H), f32)

  def wall(Wf, Ws):
    return jnp.concatenate(
        [Wf[:_H], Ws[:_H], Wf[_H:2 * _H], Ws[_H:2 * _H]], axis=1)

  w1, w2, w3 = wall(Wf1, Ws1), wall(Wf2, Ws2), wall(Wf3, Ws3)
  we_all = jnp.concatenate(
      [Wf1[2 * _H:], Ws1[2 * _H:], Wf2[2 * _H:], Ws2[2 * _H:],
       Wf3[2 * _H:], Ws3[2 * _H:]], axis=1)
  be_all = jnp.concatenate([bf1, bs1, bf2, bs2, bf3, bs3]).reshape(1, 6 * _H)

  c1f, c1s, c2f, c2s, c3f, c3s = _cprep(edge_attr, we_all, be_all)

  h0, pdf, pds, psf, pss = _proj1(x2, emb_pad, w1)
  p = _edge_pass(pdf, pds, psf, pss, c1f, c1s, dst, src, zeros)
  h1, pdf, pds, psf, pss = _proj(h0, p, w2)
  p = _edge_pass(pdf, pds, psf, pss, c2f, c2s, dst, src, zeros)
  h2, pdf, pds, psf, pss = _proj(h1, p, w3)
  p = _edge_pass(pdf, pds, psf, pss, c3f, c3s, dst, src, zeros)
  return _pool(h2, p, b2, Wlin, blin.reshape(1, _H))


# speedup vs baseline: 3.1289x; 1.8724x over previous
import functools

import jax
import jax.numpy as jnp
from jax import lax
from jax.experimental import pallas as pl
from jax.experimental.pallas import tpu as pltpu
from jax.experimental.pallas import tpu_sc as plsc

_N = 10000
_E = 320000
_H = 128
_D = 32
_G = 64

_NC = 2
_NS = 16
_NW = _NC * _NS
_EPT = _E // _NW
_CE = 80
_NCH = _EPT // _CE
_NPAD = 10240
_NPT = _NPAD // _NS

_BN = 1000
_BE = 2000

_LP = (1.472065010832413e-06, 0.999847697496239, -0.4973732161580147,
       0.3157473167582865, -0.19035433673370444, 0.08269123711198781,
       -0.017414077524446427)


def _log1p_poly(t):
  acc = jnp.full_like(t, _LP[6])
  for c in (_LP[5], _LP[4], _LP[3], _LP[2], _LP[1], _LP[0]):
    acc = acc * t + c
  return acc


def _cprep_body(ea_ref, we_ref, be_ref, *c_refs):
  c = jnp.dot(ea_ref[...], we_ref[...],
              preferred_element_type=jnp.float32) + be_ref[...]
  for j in range(6):
    c_refs[j][...] = c[:, j * _H:(j + 1) * _H]


_cprep = pl.pallas_call(
    _cprep_body,
    grid=(_E // _BE,),
    in_specs=[
        pl.BlockSpec((_BE, _D), lambda i: (i, 0)),
        pl.BlockSpec((_D, 6 * _H), lambda i: (0, 0)),
        pl.BlockSpec((1, 6 * _H), lambda i: (0, 0)),
    ],
    out_specs=[pl.BlockSpec((_BE, _H), lambda i: (i, 0))] * 6,
    out_shape=[jax.ShapeDtypeStruct((_E, _H), jnp.float32)] * 6,
)


def _proj1_body(x_ref, emb_ref, w_ref, h_ref, *p_refs):
  iot = lax.broadcasted_iota(jnp.int32, (_BN, _H), 1)
  oh = (x_ref[...] == iot).astype(jnp.float32)
  h = jnp.dot(oh, emb_ref[...], preferred_element_type=jnp.float32)
  h_ref[...] = h
  r = jnp.dot(h, w_ref[...], preferred_element_type=jnp.float32)
  for j in range(4):
    p_refs[j][...] = r[:, j * _H:(j + 1) * _H]


_proj1 = pl.pallas_call(
    _proj1_body,
    grid=(_N // _BN,),
    in_specs=[
        pl.BlockSpec((_BN, 1), lambda i: (i, 0)),
        pl.BlockSpec((_H, _H), lambda i: (0, 0)),
        pl.BlockSpec((_H, 4 * _H), lambda i: (0, 0)),
    ],
    out_specs=[pl.BlockSpec((_BN, _H), lambda i: (i, 0))] * 5,
    out_shape=[jax.ShapeDtypeStruct((_N, _H), jnp.float32)] * 5,
)


def _proj_body(h_ref, p_ref, w_ref, hn_ref, *p_refs):
  hn = jnp.maximum(h_ref[...] + p_ref[0] + p_ref[1], 0.0)
  hn_ref[...] = hn
  r = jnp.dot(hn, w_ref[...], preferred_element_type=jnp.float32)
  for j in range(4):
    p_refs[j][...] = r[:, j * _H:(j + 1) * _H]


_proj = pl.pallas_call(
    _proj_body,
    grid=(_N // _BN,),
    in_specs=[
        pl.BlockSpec((_BN, _H), lambda i: (i, 0)),
        pl.BlockSpec((_NC, _BN, _H), lambda i: (0, i, 0)),
        pl.BlockSpec((_H, 4 * _H), lambda i: (0, 0)),
    ],
    out_specs=[pl.BlockSpec((_BN, _H), lambda i: (i, 0))] * 5,
    out_shape=[jax.ShapeDtypeStruct((_N, _H), jnp.float32)] * 5,
)


def _pool_body(h_ref, p_ref, b_ref, wl_ref, bl_ref, o_ref, acc, cnt):
  i = pl.program_id(0)

  @pl.when(i == 0)
  def _():
    acc[...] = jnp.zeros_like(acc)
    cnt[...] = jnp.zeros_like(cnt)

  h3 = jnp.maximum(h_ref[...] + p_ref[0] + p_ref[1], 0.0)
  iot = lax.broadcasted_iota(jnp.int32, (_G, _BN), 0)
  pt = (b_ref[0] == iot).astype(jnp.float32)
  acc[...] += jnp.dot(pt, h3, preferred_element_type=jnp.float32)
  cnt[...] += jnp.dot(pt, jnp.ones_like(h3), preferred_element_type=jnp.float32)

  @pl.when(i == pl.num_programs(0) - 1)
  def _():
    pooled = acc[...] / jnp.maximum(cnt[...], 1.0)
    o_ref[...] = jnp.dot(pooled, wl_ref[...],
                         preferred_element_type=jnp.float32) + bl_ref[...]


_pool = pl.pallas_call(
    _pool_body,
    grid=(_N // _BN,),
    in_specs=[
        pl.BlockSpec((_BN, _H), lambda i: (i, 0)),
        pl.BlockSpec((_NC, _BN, _H), lambda i: (0, i, 0)),
        pl.BlockSpec((1, 1, _BN), lambda i: (i, 0, 0)),
        pl.BlockSpec((_H, _H), lambda i: (0, 0)),
        pl.BlockSpec((1, _H), lambda i: (0, 0)),
    ],
    out_specs=pl.BlockSpec((_G, _H), lambda i: (0, 0)),
    out_shape=jax.ShapeDtypeStruct((_G, _H), jnp.float32),
    scratch_shapes=[
        pltpu.VMEM((_G, _H), jnp.float32),
        pltpu.VMEM((_G, _H), jnp.float32),
    ],
)


def _edge_body(pdf_hbm, pds_hbm, psf_hbm, pss_hbm, cf_hbm, cs_hbm,
               dst_hbm, src_hbm, zero_hbm, out_hbm,
               agg, dstb, srcb, bbf, bbs, semf, semg):
  cid = lax.axis_index("c")
  sid = lax.axis_index("s")
  wid = sid * _NC + cid
  base = wid * _EPT

  pltpu.sync_copy(zero_hbm, agg.at[pl.ds(sid * _NPT, _NPT)])
  plsc.subcore_barrier()

  def front(i, b):
    off = base + i * _CE
    pltpu.async_copy(dst_hbm.at[pl.ds(off, _CE)], dstb.at[b], semf.at[b])
    pltpu.async_copy(src_hbm.at[pl.ds(off, _CE)], srcb.at[b], semf.at[b])
    pltpu.async_copy(cf_hbm.at[pl.ds(off, _CE)], bbf.at[b], semf.at[b])
    pltpu.async_copy(cs_hbm.at[pl.ds(off, _CE)], bbs.at[b], semf.at[b])

  def wait_front(i, b):
    off = base + i * _CE
    pltpu.make_async_copy(dst_hbm.at[pl.ds(off, _CE)], dstb.at[b],
                          semf.at[b]).wait()
    pltpu.make_async_copy(src_hbm.at[pl.ds(off, _CE)], srcb.at[b],
                          semf.at[b]).wait()
    pltpu.make_async_copy(cf_hbm.at[pl.ds(off, _CE)], bbf.at[b],
                          semf.at[b]).wait()
    pltpu.make_async_copy(cs_hbm.at[pl.ds(off, _CE)], bbs.at[b],
                          semf.at[b]).wait()

  def gathers(b):
    pltpu.async_copy(pdf_hbm.at[dstb.at[b]], bbf.at[b], semg.at[b], add=True)
    pltpu.async_copy(psf_hbm.at[srcb.at[b]], bbf.at[b], semg.at[b], add=True)
    pltpu.async_copy(pds_hbm.at[dstb.at[b]], bbs.at[b], semg.at[b], add=True)
    pltpu.async_copy(pss_hbm.at[srcb.at[b]], bbs.at[b], semg.at[b], add=True)

  def wait_gathers(b):
    pltpu.make_async_copy(pdf_hbm.at[dstb.at[b]], bbf.at[b],
                          semg.at[b]).wait()
    pltpu.make_async_copy(psf_hbm.at[srcb.at[b]], bbf.at[b],
                          semg.at[b]).wait()
    pltpu.make_async_copy(pds_hbm.at[dstb.at[b]], bbs.at[b],
                          semg.at[b]).wait()
    pltpu.make_async_copy(pss_hbm.at[srcb.at[b]], bbs.at[b],
                          semg.at[b]).wait()

  def compute_scatter(b):
    def edge(e, c2):
      for k in range(_H // 16):
        f = bbf[b, e, pl.ds(k * 16, 16)]
        s = bbs[b, e, pl.ds(k * 16, 16)]
        sig = 1.0 / (1.0 + jnp.exp(-f))
        t = jnp.exp(-jnp.abs(s))
        sp = jnp.maximum(s, 0.0) + _log1p_poly(t)
        bbf[b, e, pl.ds(k * 16, 16)] = sig * sp
      return c2

    lax.fori_loop(0, _CE, edge, 0)
    pltpu.sync_copy(bbf.at[b], agg.at[dstb.at[b]], add=True)

  front(0, 0)
  wait_front(0, 0)
  gathers(0)
  front(1, 1)

  def pair(p, carry):
    c0 = 2 * p

    wait_gathers(0)

    @pl.when(c0 + 1 < _NCH)
    def _():
      wait_front(c0 + 1, 1)
      gathers(1)

    compute_scatter(0)

    @pl.when(c0 + 2 < _NCH)
    def _():
      front(c0 + 2, 0)

    @pl.when(c0 + 1 < _NCH)
    def _():
      wait_gathers(1)

      @pl.when(c0 + 2 < _NCH)
      def _():
        wait_front(c0 + 2, 0)
        gathers(0)

      compute_scatter(1)

      @pl.when(c0 + 3 < _NCH)
      def _():
        front(c0 + 3, 1)

    return carry

  lax.fori_loop(0, (_NCH + 1) // 2, pair, 0)
  plsc.subcore_barrier()
  pltpu.sync_copy(agg.at[pl.ds(sid * _NPT, _NPT)],
                  out_hbm.at[cid, pl.ds(sid * _NPT, _NPT)])


@functools.cache
def _edge_pass_fn():
  return functools.partial(
      pl.kernel,
      out_type=jax.ShapeDtypeStruct((_NC, _NPAD, _H), jnp.float32),
      mesh=plsc.VectorSubcoreMesh(core_axis_name="c", subcore_axis_name="s",
                                  num_cores=_NC, num_subcores=_NS),
      scratch_types=[
          pltpu.VMEM_SHARED((_NPAD, _H), jnp.float32),
          pltpu.VMEM((2, _CE), jnp.int32),
          pltpu.VMEM((2, _CE), jnp.int32),
          pltpu.VMEM((2, _CE, _H), jnp.float32),
          pltpu.VMEM((2, _CE, _H), jnp.float32),
          pltpu.SemaphoreType.DMA((2,)),
          pltpu.SemaphoreType.DMA((2,)),
      ],
  )(_edge_body)


def _edge_pass(*args):
  return _edge_pass_fn()(*args)


def kernel(x, edge_index, edge_attr, batch, emb,
           Wf1, bf1, Ws1, bs1, Wf2, bf2, Ws2, bs2, Wf3, bf3, Ws3, bs3,
           Wlin, blin):
  f32 = jnp.float32
  x2 = x.astype(jnp.int32).reshape(_N, 1)
  src = edge_index[0].astype(jnp.int32)
  dst = edge_index[1].astype(jnp.int32)
  b2 = batch.astype(jnp.int32).reshape(_N // _BN, 1, _BN)
  emb_pad = jnp.zeros((_H, _H), f32).at[:emb.shape[0]].set(emb)
  zeros = jnp.zeros((_NPT, _H), f32)

  def wall(Wf, Ws):
    return jnp.concatenate(
        [Wf[:_H], Ws[:_H], Wf[_H:2 * _H], Ws[_H:2 * _H]], axis=1)

  w1, w2, w3 = wall(Wf1, Ws1), wall(Wf2, Ws2), wall(Wf3, Ws3)
  we_all = jnp.concatenate(
      [Wf1[2 * _H:], Ws1[2 * _H:], Wf2[2 * _H:], Ws2[2 * _H:],
       Wf3[2 * _H:], Ws3[2 * _H:]], axis=1)
  be_all = jnp.concatenate([bf1, bs1, bf2, bs2, bf3, bs3]).reshape(1, 6 * _H)

  c1f, c1s, c2f, c2s, c3f, c3s = _cprep(edge_attr, we_all, be_all)

  h0, pdf, pds, psf, pss = _proj1(x2, emb_pad, w1)
  p = _edge_pass(pdf, pds, psf, pss, c1f, c1s, dst, src, zeros)
  h1, pdf, pds, psf, pss = _proj(h0, p, w2)
  p = _edge_pass(pdf, pds, psf, pss, c2f, c2s, dst, src, zeros)
  h2, pdf, pds, psf, pss = _proj(h1, p, w3)
  p = _edge_pass(pdf, pds, psf, pss, c3f, c3s, dst, src, zeros)
  return _pool(h2, p, b2, Wlin, blin.reshape(1, _H))
